# baseline (speedup 1.0000x reference)
import functools

import jax
import jax.numpy as jnp
from jax import lax
from jax.experimental import pallas as pl
from jax.experimental.pallas import tpu as pltpu
from jax.experimental.pallas import tpu_sc as plsc

EMBED = 64
BATCH = 4096
SEQ = 200

NUM_CORES = 2
NUM_SUBCORES = 16
NUM_WORKERS = NUM_CORES * NUM_SUBCORES
BPW = BATCH // NUM_WORKERS

CHUNKS = ((0, 104), (104, 96))
EV = EMBED // 16
UNROLL = 8
assert SEQ % UNROLL == 0


def _sc_pooled_sums(x, table):
    mesh = plsc.VectorSubcoreMesh(
        core_axis_name="c", subcore_axis_name="s", num_cores=NUM_CORES
    )

    @functools.partial(
        pl.kernel,
        mesh=mesh,
        compiler_params=pltpu.CompilerParams(use_tc_tiling_on_sc=False),
        out_type=jax.ShapeDtypeStruct((BATCH, EMBED), jnp.float32),
        scratch_types=[
            pltpu.VMEM((BPW * SEQ,), jnp.int32),
            pltpu.VMEM((SEQ, EMBED), jnp.float32),
            pltpu.VMEM((SEQ, EMBED), jnp.float32),
            pltpu.VMEM((BPW, EMBED), jnp.float32),
            pltpu.SemaphoreType.DMA,
            pltpu.SemaphoreType.DMA,
        ],
    )
    def k(x_hbm, table_hbm, out_hbm, idx_v, buf0, buf1, pooled, sem0, sem1):
        wid = lax.axis_index("s") * NUM_CORES + lax.axis_index("c")
        base = wid * BPW
        pltpu.sync_copy(x_hbm.at[pl.ds(base * SEQ, BPW * SEQ)], idx_v)

        def issue(b, buf, sem):
            for off, ln in CHUNKS:
                pltpu.async_copy(
                    table_hbm.at[idx_v.at[pl.ds(b * SEQ + off, ln)]],
                    buf.at[pl.ds(off, ln)],
                    sem,
                )

        def wait(buf, sem):
            pltpu.make_async_copy(
                table_hbm.at[idx_v.at[pl.ds(0, SEQ)]], buf, sem
            ).wait()

        def compute(b, buf):
            def body(i, acc):
                accs = list(acc)
                for u in range(UNROLL):
                    s = i * UNROLL + u
                    for e in range(EV):
                        v = buf[s, pl.ds(e * 16, 16)]
                        accs[e] = accs[e] + jnp.maximum(v, 0.01 * v)
                return tuple(accs)

            zero = jnp.zeros((16,), jnp.float32)
            acc = lax.fori_loop(0, SEQ // UNROLL, body, (zero,) * EV)
            for e in range(EV):
                pooled[b, pl.ds(e * 16, 16)] = acc[e]

        issue(0, buf0, sem0)

        def step(j, carry):
            b = 2 * j
            issue(b + 1, buf1, sem1)
            wait(buf0, sem0)
            compute(b, buf0)
            issue(b + 2, buf0, sem0)
            wait(buf1, sem1)
            compute(b + 1, buf1)
            return carry

        lax.fori_loop(0, BPW // 2 - 1, step, 0)
        issue(BPW - 1, buf1, sem1)
        wait(buf0, sem0)
        compute(BPW - 2, buf0)
        wait(buf1, sem1)
        compute(BPW - 1, buf1)
        pltpu.sync_copy(pooled, out_hbm.at[pl.ds(base, BPW)])

    return k(x, table)


def _tc_mlp(sums, W1, b1, W2, b2):
    def body(s_ref, w1_ref, b1_ref, w2_ref, b2_ref, o_ref):
        pooled = s_ref[...] * (1.0 / SEQ)
        h = lax.dot_general(
            pooled, w1_ref[...], (((1,), (1,)), ((), ())),
            preferred_element_type=jnp.float32,
        ) + b1_ref[...]
        h = jnp.where(h >= 0, h, 0.01 * h)
        logit = jnp.sum(h * w2_ref[...], axis=1, keepdims=True) + b2_ref[0, 0]
        o_ref[...] = jax.nn.sigmoid(logit)

    out = pl.pallas_call(
        body,
        out_shape=jax.ShapeDtypeStruct((BATCH, 1), jnp.float32),
    )(sums, W1, b1, W2, b2.reshape(1, 1))
    return jnp.squeeze(out, -1)


def kernel(x, table, W1, b1, W2, b2):
    x = x.astype(jnp.int32).reshape(-1)
    sums = _sc_pooled_sums(x, table)
    return _tc_mlp(sums, W1, b1, W2, b2)
